# baseline (speedup 1.0000x reference)
import functools

import jax
import jax.numpy as jnp
from jax import lax
from jax.experimental import pallas as pl
from jax.experimental.pallas import tpu as pltpu
from jax.experimental.pallas import tpu_sc as plsc

_CONF = 0.25
_IOU = 0.45
_MAXWH = 4096.0
_MAXDET = 300
_B = 4
_N = 20000
_NC = 80
_F = 85
_W = 96
_CH = 512
_K = 1024
_NPAD = 20480
_KPAD = 304
_OUTW = 16
_PREP_ROWS = 2000


def _prep_body(x_ref, xp_ref, score_ref):
    xb = x_ref[...]
    conf0 = xb[:, 4:5]
    cls = xb[:, 5:_F] * conf0
    conf = jnp.max(cls, axis=1, keepdims=True)
    valid = (conf0 > _CONF) & (conf > _CONF)
    score = jnp.where(valid, conf, -jnp.inf)
    pad = jnp.zeros((_PREP_ROWS, _W - _F - 1), jnp.float32)
    xp_ref[...] = jnp.concatenate([xb, score, pad], axis=1)
    score_ref[...] = score.reshape(1, 1, _PREP_ROWS)


def _prep(flat):
    nblk = _B * _N // _PREP_ROWS
    return pl.pallas_call(
        _prep_body,
        grid=(nblk,),
        in_specs=[pl.BlockSpec((_PREP_ROWS, _F), lambda i: (i, 0))],
        out_specs=[
            pl.BlockSpec((_PREP_ROWS, _W), lambda i: (i, 0)),
            pl.BlockSpec((1, 1, _PREP_ROWS), lambda i: (i, 0, 0)),
        ],
        out_shape=[
            jax.ShapeDtypeStruct((_B * _N, _W), jnp.float32),
            jax.ShapeDtypeStruct((nblk, 1, _PREP_ROWS), jnp.float32),
        ],
    )(flat)


_SC_MESH = plsc.VectorSubcoreMesh(core_axis_name="c", subcore_axis_name="s")


def _make_nms(npad, nlimit):
    nchunks = npad // _CH

    @functools.partial(
        pl.kernel,
        out_type=jax.ShapeDtypeStruct((_B, _KPAD * _OUTW), jnp.float32),
        mesh=_SC_MESH,
        scratch_types=[
            pltpu.VMEM((npad,), jnp.int32),
            pltpu.VMEM((_CH, _W), jnp.float32),
            pltpu.VMEM((_KPAD,), jnp.float32),
            pltpu.VMEM((_KPAD,), jnp.float32),
            pltpu.VMEM((_KPAD,), jnp.float32),
            pltpu.VMEM((_KPAD,), jnp.float32),
            pltpu.VMEM((_KPAD,), jnp.float32),
            pltpu.VMEM((_KPAD * _OUTW,), jnp.float32),
            pltpu.SMEM((1,), jnp.int32),
            pltpu.SMEM((1,), jnp.int32),
            pltpu.SemaphoreType.DMA,
        ],
        compiler_params=pltpu.CompilerParams(
            needs_layout_passes=False, use_tc_tiling_on_sc=False),
    )
    def _nms_sc(tab_hbm, ord_hbm, out_hbm,
                idx_v, chunk_v, kx1, ky1, kx2, ky2, ka, outb,
                cnt_ref, go_ref, sem):
        img = lax.axis_index("s") * 2 + lax.axis_index("c")

        @pl.when(img < _B)
        def _run():
            pltpu.sync_copy(ord_hbm.at[img], idx_v)

            zero16 = jnp.zeros((16,), jnp.float32)
            pinf16 = jnp.full((16,), jnp.inf, jnp.float32)
            ninf16 = jnp.full((16,), -jnp.inf, jnp.float32)
            lanes = lax.broadcasted_iota(jnp.int32, (16,), 0)
            lane0 = lanes == 0

            def _init_out(r, carry):
                outb[pl.ds(pl.multiple_of(r * 16, 16), 16)] = zero16
                return carry

            lax.fori_loop(0, _KPAD, _init_out, 0)

            def _init_kept(s, carry):
                o = pl.multiple_of(s * 16, 16)
                kx1[pl.ds(o, 16)] = pinf16
                ky1[pl.ds(o, 16)] = pinf16
                kx2[pl.ds(o, 16)] = ninf16
                ky2[pl.ds(o, 16)] = ninf16
                ka[pl.ds(o, 16)] = zero16
                return carry

            lax.fori_loop(0, _KPAD // 16, _init_kept, 0)

            cnt_ref[0] = 0
            go_ref[0] = 1

            dnums = lax.GatherDimensionNumbers(
                offset_dims=(), collapsed_slice_dims=(0,),
                start_index_map=(0,))

            def _gath(vec, idx):
                return lax.gather(
                    vec, idx.reshape(16, 1), dnums, slice_sizes=(1,),
                    mode=lax.GatherScatterMode.PROMISE_IN_BOUNDS)

            def _bcast(vec, k):
                return _gath(vec, jnp.full((16,), k, jnp.int32))

            rotidx = [(lanes + k) % 16 for k in (8, 4, 2, 1)]

            def _chunk(c, carry):
                @pl.when(go_ref[0] == 1)
                def _do_chunk():
                    src = tab_hbm.at[idx_v.at[pl.ds(
                        pl.multiple_of(c * _CH, _CH), _CH)]]
                    pltpu.async_copy(src, chunk_v, sem).wait()
                    trip = jnp.minimum(_CH, nlimit - c * _CH)

                    def _cand(local, carry2):
                        @pl.when(go_ref[0] == 1)
                        def _do_cand():
                            vecs = [
                                chunk_v[local, pl.ds(16 * s, 16)]
                                for s in range(6)
                            ]
                            go1 = jnp.any((vecs[5] > _CONF) & (lanes == 5))

                            conf0v = _bcast(vecs[0], 4)
                            scv = _bcast(vecs[5], 5)

                            mprods = []
                            gidx = []
                            for s in range(6):
                                gi = lanes + (16 * s)
                                m = (gi >= 5) & (gi < _F)
                                mprods.append(jnp.where(
                                    m, vecs[s] * conf0v, -jnp.inf))
                                gidx.append(gi)
                            mx = mprods[0]
                            for s in range(1, 6):
                                mx = jnp.maximum(mx, mprods[s])
                            gmax = mx
                            for ridx in rotidx:
                                gmax = jnp.maximum(gmax, _gath(gmax, ridx))
                            jcand = jnp.full((16,), 4096.0, jnp.float32)
                            for s in range(6):
                                jc = (gidx[s] - 5).astype(jnp.float32)
                                jcand = jnp.minimum(
                                    jcand,
                                    jnp.where(mprods[s] == gmax, jc, 4096.0))
                            for ridx in rotidx:
                                jcand = jnp.minimum(jcand, _gath(jcand, ridx))
                            jf = jcand

                            cxv = _bcast(vecs[0], 0)
                            cyv = _bcast(vecs[0], 1)
                            wv = _bcast(vecs[0], 2)
                            hv = _bcast(vecs[0], 3)
                            w2 = wv / 2.0
                            h2 = hv / 2.0
                            bx1 = cxv - w2
                            by1 = cyv - h2
                            bx2 = cxv + w2
                            by2 = cyv + h2
                            off = jf * _MAXWH
                            vx1 = bx1 + off
                            vy1 = by1 + off
                            vx2 = bx2 + off
                            vy2 = by2 + off
                            vai = (vx2 - vx1) * (vy2 - vy1)

                            cnt = cnt_ref[0]
                            nsl = (cnt + 15) // 16

                            def _scan(s, acc):
                                o = pl.multiple_of(s * 16, 16)
                                gx1 = kx1[pl.ds(o, 16)]
                                gy1 = ky1[pl.ds(o, 16)]
                                gx2 = kx2[pl.ds(o, 16)]
                                gy2 = ky2[pl.ds(o, 16)]
                                ga = ka[pl.ds(o, 16)]
                                xx1 = jnp.maximum(vx1, gx1)
                                yy1 = jnp.maximum(vy1, gy1)
                                xx2 = jnp.minimum(vx2, gx2)
                                yy2 = jnp.minimum(vy2, gy2)
                                inter = jnp.maximum(xx2 - xx1, 0.0) * (
                                    jnp.maximum(yy2 - yy1, 0.0))
                                iou = inter / (ga + vai - inter + 1e-9)
                                return acc | (iou > _IOU)

                            supb = lax.fori_loop(
                                0, nsl, _scan, jnp.zeros((16,), jnp.bool_))
                            keep = go1 & jnp.logical_not(jnp.any(supb))

                            @pl.when(keep)
                            def _append():
                                cnt16 = jnp.full((16,), cnt, jnp.int32)
                                plsc.store_scatter(
                                    kx1, [cnt16], vx1, mask=lane0)
                                plsc.store_scatter(
                                    ky1, [cnt16], vy1, mask=lane0)
                                plsc.store_scatter(
                                    kx2, [cnt16], vx2, mask=lane0)
                                plsc.store_scatter(
                                    ky2, [cnt16], vy2, mask=lane0)
                                plsc.store_scatter(
                                    ka, [cnt16], vai, mask=lane0)
                                outrow = jnp.where(
                                    lanes == 0, bx1, jnp.where(
                                        lanes == 1, by1, jnp.where(
                                            lanes == 2, bx2, jnp.where(
                                                lanes == 3, by2, jnp.where(
                                                    lanes == 4, scv,
                                                    jnp.where(
                                                        lanes == 5, jf,
                                                        zero16))))))
                                outb[pl.ds(
                                    pl.multiple_of(cnt * 16, 16), 16)] = (
                                    outrow)

                            cnt2 = cnt + keep.astype(jnp.int32)
                            cnt_ref[0] = cnt2
                            go_ref[0] = (
                                go1 & (cnt2 < _MAXDET)).astype(jnp.int32)

                        return carry2

                    lax.fori_loop(0, trip, _cand, 0)

                return carry

            lax.fori_loop(0, nchunks, _chunk, 0)

            outb[pl.ds(pl.multiple_of(_MAXDET * 16, 16), 16)] = jnp.full(
                (16,), go_ref[0].astype(jnp.float32))

            pltpu.sync_copy(outb, out_hbm.at[img])

    return _nms_sc


_nms_fast = _make_nms(_K, _K)
_nms_full = _make_nms(_NPAD, _N)


def kernel(x):
    flat = x.reshape(_B * _N, _F)
    xp, score3 = _prep(flat)
    score = score3.reshape(_B, _N)
    base = (jnp.arange(_B, dtype=jnp.int32) * _N)[:, None]

    kidx = lax.top_k(score, _K)[1].astype(jnp.int32)
    out_fast = _nms_fast(xp, kidx + base)
    need_full = jnp.any(
        out_fast.reshape(_B, _KPAD, _OUTW)[:, _MAXDET, 0] > 0.5)

    def _full(_):
        order = jnp.argsort(-score, axis=1).astype(jnp.int32)
        ofs = jnp.pad(order + base, ((0, 0), (0, _NPAD - _N)))
        return _nms_full(xp, ofs)

    out = lax.cond(need_full, _full, lambda _: out_fast, None)
    return out.reshape(_B, _KPAD, _OUTW)[:, :_MAXDET, :6]


# speedup vs baseline: 1.0018x; 1.0018x over previous
import functools

import jax
import jax.numpy as jnp
from jax import lax
from jax.experimental import pallas as pl
from jax.experimental.pallas import tpu as pltpu
from jax.experimental.pallas import tpu_sc as plsc

_CONF = 0.25
_IOU = 0.45
_MAXWH = 4096.0
_MAXDET = 300
_B = 4
_N = 20000
_NC = 80
_F = 85
_W = 96
_CH = 512
_K = 512
_NPAD = 20480
_KPAD = 304
_OUTW = 16
_PREP_ROWS = 2000


def _prep_body(x_ref, xp_ref, score_ref):
    xb = x_ref[...]
    conf0 = xb[:, 4:5]
    cls = xb[:, 5:_F] * conf0
    conf = jnp.max(cls, axis=1, keepdims=True)
    valid = (conf0 > _CONF) & (conf > _CONF)
    score = jnp.where(valid, conf, -jnp.inf)
    pad = jnp.zeros((_PREP_ROWS, _W - _F - 1), jnp.float32)
    xp_ref[...] = jnp.concatenate([xb, score, pad], axis=1)
    score_ref[...] = score.reshape(1, 1, _PREP_ROWS)


def _prep(flat):
    nblk = _B * _N // _PREP_ROWS
    return pl.pallas_call(
        _prep_body,
        grid=(nblk,),
        in_specs=[pl.BlockSpec((_PREP_ROWS, _F), lambda i: (i, 0))],
        out_specs=[
            pl.BlockSpec((_PREP_ROWS, _W), lambda i: (i, 0)),
            pl.BlockSpec((1, 1, _PREP_ROWS), lambda i: (i, 0, 0)),
        ],
        out_shape=[
            jax.ShapeDtypeStruct((_B * _N, _W), jnp.float32),
            jax.ShapeDtypeStruct((nblk, 1, _PREP_ROWS), jnp.float32),
        ],
    )(flat)


_SC_MESH = plsc.VectorSubcoreMesh(core_axis_name="c", subcore_axis_name="s")


def _make_nms(npad, nlimit):
    nchunks = npad // _CH

    @functools.partial(
        pl.kernel,
        out_type=jax.ShapeDtypeStruct((_B, _KPAD * _OUTW), jnp.float32),
        mesh=_SC_MESH,
        scratch_types=[
            pltpu.VMEM((npad,), jnp.int32),
            pltpu.VMEM((_CH, _W), jnp.float32),
            pltpu.VMEM((_KPAD,), jnp.float32),
            pltpu.VMEM((_KPAD,), jnp.float32),
            pltpu.VMEM((_KPAD,), jnp.float32),
            pltpu.VMEM((_KPAD,), jnp.float32),
            pltpu.VMEM((_KPAD,), jnp.float32),
            pltpu.VMEM((_KPAD * _OUTW,), jnp.float32),
            pltpu.SMEM((1,), jnp.int32),
            pltpu.SMEM((1,), jnp.int32),
            pltpu.SemaphoreType.DMA,
        ],
        compiler_params=pltpu.CompilerParams(
            needs_layout_passes=False, use_tc_tiling_on_sc=False),
    )
    def _nms_sc(tab_hbm, ord_hbm, out_hbm,
                idx_v, chunk_v, kx1, ky1, kx2, ky2, ka, outb,
                cnt_ref, go_ref, sem):
        img = lax.axis_index("s") * 2 + lax.axis_index("c")

        @pl.when(img < _B)
        def _run():
            pltpu.sync_copy(ord_hbm.at[img], idx_v)

            zero16 = jnp.zeros((16,), jnp.float32)
            pinf16 = jnp.full((16,), jnp.inf, jnp.float32)
            ninf16 = jnp.full((16,), -jnp.inf, jnp.float32)
            lanes = lax.broadcasted_iota(jnp.int32, (16,), 0)
            lane0 = lanes == 0

            def _init_out(r, carry):
                outb[pl.ds(pl.multiple_of(r * 16, 16), 16)] = zero16
                return carry

            lax.fori_loop(0, _KPAD, _init_out, 0)

            def _init_kept(s, carry):
                o = pl.multiple_of(s * 16, 16)
                kx1[pl.ds(o, 16)] = pinf16
                ky1[pl.ds(o, 16)] = pinf16
                kx2[pl.ds(o, 16)] = ninf16
                ky2[pl.ds(o, 16)] = ninf16
                ka[pl.ds(o, 16)] = zero16
                return carry

            lax.fori_loop(0, _KPAD // 16, _init_kept, 0)

            cnt_ref[0] = 0
            go_ref[0] = 1

            dnums = lax.GatherDimensionNumbers(
                offset_dims=(), collapsed_slice_dims=(0,),
                start_index_map=(0,))

            def _gath(vec, idx):
                return lax.gather(
                    vec, idx.reshape(16, 1), dnums, slice_sizes=(1,),
                    mode=lax.GatherScatterMode.PROMISE_IN_BOUNDS)

            def _bcast(vec, k):
                return _gath(vec, jnp.full((16,), k, jnp.int32))

            rotidx = [(lanes + k) % 16 for k in (8, 4, 2, 1)]

            def _chunk(c, carry):
                @pl.when(go_ref[0] == 1)
                def _do_chunk():
                    src = tab_hbm.at[idx_v.at[pl.ds(
                        pl.multiple_of(c * _CH, _CH), _CH)]]
                    pltpu.async_copy(src, chunk_v, sem).wait()
                    trip = jnp.minimum(_CH, nlimit - c * _CH)

                    def _cand(local, carry2):
                        @pl.when(go_ref[0] == 1)
                        def _do_cand():
                            vecs = [
                                chunk_v[local, pl.ds(16 * s, 16)]
                                for s in range(6)
                            ]
                            go1 = jnp.any((vecs[5] > _CONF) & (lanes == 5))

                            conf0v = _bcast(vecs[0], 4)
                            scv = _bcast(vecs[5], 5)

                            mprods = []
                            gidx = []
                            for s in range(6):
                                gi = lanes + (16 * s)
                                m = (gi >= 5) & (gi < _F)
                                mprods.append(jnp.where(
                                    m, vecs[s] * conf0v, -jnp.inf))
                                gidx.append(gi)
                            mx = mprods[0]
                            for s in range(1, 6):
                                mx = jnp.maximum(mx, mprods[s])
                            gmax = mx
                            for ridx in rotidx:
                                gmax = jnp.maximum(gmax, _gath(gmax, ridx))
                            jcand = jnp.full((16,), 4096.0, jnp.float32)
                            for s in range(6):
                                jc = (gidx[s] - 5).astype(jnp.float32)
                                jcand = jnp.minimum(
                                    jcand,
                                    jnp.where(mprods[s] == gmax, jc, 4096.0))
                            for ridx in rotidx:
                                jcand = jnp.minimum(jcand, _gath(jcand, ridx))
                            jf = jcand

                            cxv = _bcast(vecs[0], 0)
                            cyv = _bcast(vecs[0], 1)
                            wv = _bcast(vecs[0], 2)
                            hv = _bcast(vecs[0], 3)
                            w2 = wv / 2.0
                            h2 = hv / 2.0
                            bx1 = cxv - w2
                            by1 = cyv - h2
                            bx2 = cxv + w2
                            by2 = cyv + h2
                            off = jf * _MAXWH
                            vx1 = bx1 + off
                            vy1 = by1 + off
                            vx2 = bx2 + off
                            vy2 = by2 + off
                            vai = (vx2 - vx1) * (vy2 - vy1)

                            cnt = cnt_ref[0]
                            nsl = (cnt + 15) // 16

                            def _scan(s, acc):
                                o = pl.multiple_of(s * 16, 16)
                                gx1 = kx1[pl.ds(o, 16)]
                                gy1 = ky1[pl.ds(o, 16)]
                                gx2 = kx2[pl.ds(o, 16)]
                                gy2 = ky2[pl.ds(o, 16)]
                                ga = ka[pl.ds(o, 16)]
                                xx1 = jnp.maximum(vx1, gx1)
                                yy1 = jnp.maximum(vy1, gy1)
                                xx2 = jnp.minimum(vx2, gx2)
                                yy2 = jnp.minimum(vy2, gy2)
                                inter = jnp.maximum(xx2 - xx1, 0.0) * (
                                    jnp.maximum(yy2 - yy1, 0.0))
                                iou = inter / (ga + vai - inter + 1e-9)
                                return acc | (iou > _IOU)

                            supb = lax.fori_loop(
                                0, nsl, _scan, jnp.zeros((16,), jnp.bool_))
                            keep = go1 & jnp.logical_not(jnp.any(supb))

                            @pl.when(keep)
                            def _append():
                                cnt16 = jnp.full((16,), cnt, jnp.int32)
                                plsc.store_scatter(
                                    kx1, [cnt16], vx1, mask=lane0)
                                plsc.store_scatter(
                                    ky1, [cnt16], vy1, mask=lane0)
                                plsc.store_scatter(
                                    kx2, [cnt16], vx2, mask=lane0)
                                plsc.store_scatter(
                                    ky2, [cnt16], vy2, mask=lane0)
                                plsc.store_scatter(
                                    ka, [cnt16], vai, mask=lane0)
                                outrow = jnp.where(
                                    lanes == 0, bx1, jnp.where(
                                        lanes == 1, by1, jnp.where(
                                            lanes == 2, bx2, jnp.where(
                                                lanes == 3, by2, jnp.where(
                                                    lanes == 4, scv,
                                                    jnp.where(
                                                        lanes == 5, jf,
                                                        zero16))))))
                                outb[pl.ds(
                                    pl.multiple_of(cnt * 16, 16), 16)] = (
                                    outrow)

                            cnt2 = cnt + keep.astype(jnp.int32)
                            cnt_ref[0] = cnt2
                            go_ref[0] = (
                                go1 & (cnt2 < _MAXDET)).astype(jnp.int32)

                        return carry2

                    lax.fori_loop(0, trip, _cand, 0)

                return carry

            lax.fori_loop(0, nchunks, _chunk, 0)

            outb[pl.ds(pl.multiple_of(_MAXDET * 16, 16), 16)] = jnp.full(
                (16,), go_ref[0].astype(jnp.float32))

            pltpu.sync_copy(outb, out_hbm.at[img])

    return _nms_sc


_nms_fast = _make_nms(_K, _K)
_nms_full = _make_nms(_NPAD, _N)


def kernel(x):
    flat = x.reshape(_B * _N, _F)
    xp, score3 = _prep(flat)
    score = score3.reshape(_B, _N)
    base = (jnp.arange(_B, dtype=jnp.int32) * _N)[:, None]

    kidx = lax.top_k(score, _K)[1].astype(jnp.int32)
    out_fast = _nms_fast(xp, kidx + base)
    need_full = jnp.any(
        out_fast.reshape(_B, _KPAD, _OUTW)[:, _MAXDET, 0] > 0.5)

    def _full(_):
        order = jnp.argsort(-score, axis=1).astype(jnp.int32)
        ofs = jnp.pad(order + base, ((0, 0), (0, _NPAD - _N)))
        return _nms_full(xp, ofs)

    out = lax.cond(need_full, _full, lambda _: out_fast, None)
    return out.reshape(_B, _KPAD, _OUTW)[:, :_MAXDET, :6]
